# baseline (speedup 1.0000x reference)
import functools

import jax
import jax.numpy as jnp
from jax import lax
from jax.experimental import pallas as pl
from jax.experimental.pallas import tpu as pltpu
from jax.experimental.pallas import tpu_sc as plsc

B = 16384
HB = B // 2
D = 99
N_DATASETS = 100000

NC, NS = 2, 16
NW = NC * NS
SEG0 = 51200
SEG1 = N_DATASETS - SEG0
SEGS = ((0, SEG0), (SEG0, SEG1))


def _build_sc_call():
    mesh = plsc.VectorSubcoreMesh(
        core_axis_name="c", subcore_axis_name="s",
        num_cores=NC, num_subcores=NS)

    @functools.partial(
        pl.kernel,
        mesh=mesh,
        compiler_params=pltpu.CompilerParams(needs_layout_passes=False),
        out_type=jax.ShapeDtypeStruct((D, B), jnp.float32),
        scratch_types=[
            pltpu.VMEM((SEG0,), jnp.float32),
            pltpu.VMEM((SEG1,), jnp.float32),
            pltpu.VMEM((HB,), jnp.int32),
            pltpu.VMEM((B,), jnp.float32),
            pltpu.VMEM((D + 16,), jnp.float32),
            pltpu.VMEM((D + 16,), jnp.float32),
            pltpu.SemaphoreType.DMA,
        ],
    )
    def sc_call(u_hbm, lap_hbm, idx_hbm, at_hbm, alt_hbm, out_hbm,
                buf0, buf1, idb, acc, u_v, lap_v, psem):
        wid = lax.axis_index("s") * NC + lax.axis_index("c")
        bufs = (buf0, buf1)
        pltpu.sync_copy(u_hbm, u_v.at[pl.ds(0, D)])
        pltpu.sync_copy(lap_hbm, lap_v.at[pl.ds(0, D)])

        UNROLL = 8

        def gather_stage(buf, c, seg, use_u, first, ranges):
            lo, sz = seg
            cv = u_v if use_u else lap_v
            coef = cv[pl.ds(c, 16)][0]
            for i0, n in ranges:
                pltpu.sync_copy(idx_hbm.at[pl.ds(i0, n)], idb.at[pl.ds(0, n)])

                def body(v, carry):
                    w0 = v * (16 * UNROLL)
                    for s in range(UNROLL):
                        o = w0 + s * 16
                        idvec = idb[pl.ds(o, 16)]
                        mask = (idvec >= lo) & (idvec < lo + sz)
                        loc = jnp.where(mask, idvec - lo, 0)
                        val = plsc.load_gather(buf, [loc], mask=mask)
                        contrib = jnp.where(mask, val * coef,
                                            jnp.zeros((16,), jnp.float32))
                        if not first:
                            contrib = contrib + acc[pl.ds(i0 + o, 16)]
                        acc[pl.ds(i0 + o, 16)] = contrib
                    return carry
                lax.fori_loop(0, n // (16 * UNROLL), body, 0)

        def out_write(c, ranges):
            for i0, n in ranges:
                pltpu.sync_copy(acc.at[pl.ds(i0, n)],
                                out_hbm.at[c, pl.ds(i0, n)])

        FULL = [(0, HB), (HB, HB)]
        TAIL = D - 3 * NW
        QB = B // 4
        ct = 3 * NW + wid // 4
        q = wid % 4

        seq = []
        for rnd in range(3):
            c = wid + rnd * NW
            for tbl, use_u in ((at_hbm, True), (alt_hbm, False)):
                for si, seg in enumerate(SEGS):
                    first = use_u and si == 0
                    seq.append((tbl, c, seg, use_u, first))

        def src(k):
            tbl, c, seg, _, _ = seq[k]
            return tbl.at[c, pl.ds(seg[0], seg[1])]

        L = len(seq)
        pltpu.sync_copy(src(0), bufs[0])
        pending = pltpu.async_copy(src(1), bufs[1], psem)
        for k in range(L):
            tbl, c, seg, use_u, first = seq[k]
            gather_stage(bufs[k % 2], c, seg, use_u, first, FULL)
            if k + 1 < L:
                pending.wait()
                if k + 2 < L:
                    pending = pltpu.async_copy(src(k + 2), bufs[k % 2], psem)
            if k % 4 == 3:
                out_write(c, FULL)

        @pl.when(wid < 4 * TAIL)
        def _():
            rng = [(q * QB, QB)]
            for tbl, use_u in ((at_hbm, True), (alt_hbm, False)):
                for si, seg in enumerate(SEGS):
                    pltpu.sync_copy(tbl.at[ct, pl.ds(seg[0], seg[1])],
                                    bufs[si])
                    gather_stage(bufs[si], ct, seg, use_u,
                                 use_u and si == 0, rng)
            out_write(ct, rng)

    return sc_call


_SC_CALL = None


def kernel(x, data_id, frame, a_, b_, alpha_):
    global _SC_CALL
    if _SC_CALL is None:
        _SC_CALL = _build_sc_call()
    u = x[:, 0]
    lap = x[:, 2]
    idx = data_id.astype(jnp.int32)
    out_t = _SC_CALL(u, lap, idx, a_.T, alpha_.T)
    return out_t.T


# speedup vs baseline: 1.4705x; 1.4705x over previous
import functools

import jax
import jax.numpy as jnp
from jax import lax
from jax.experimental import pallas as pl
from jax.experimental.pallas import tpu as pltpu
from jax.experimental.pallas import tpu_sc as plsc

B = 16384
HB = B // 2
D = 99
N_DATASETS = 100000

NC, NS = 2, 16
NW = NC * NS
ROUNDS = (D + NW - 1) // NW


def _build_sc_call():
    mesh = plsc.VectorSubcoreMesh(
        core_axis_name="c", subcore_axis_name="s",
        num_cores=NC, num_subcores=NS)

    @functools.partial(
        pl.kernel,
        mesh=mesh,
        compiler_params=pltpu.CompilerParams(needs_layout_passes=False),
        out_type=jax.ShapeDtypeStruct((D, B), jnp.float32),
        scratch_types=[
            pltpu.VMEM((N_DATASETS,), jnp.float32),
            pltpu.VMEM((HB,), jnp.int32),
            pltpu.VMEM((B,), jnp.float32),
            pltpu.VMEM((D + 16,), jnp.float32),
            pltpu.VMEM((D + 16,), jnp.float32),
        ],
    )
    def sc_call(u_hbm, lap_hbm, idx_hbm, at_hbm, alt_hbm, out_hbm,
                rowb, idb, acc, u_v, lap_v):
        wid = lax.axis_index("s") * NC + lax.axis_index("c")
        pltpu.sync_copy(u_hbm, u_v.at[pl.ds(0, D)])
        pltpu.sync_copy(lap_hbm, lap_v.at[pl.ds(0, D)])

        UNROLL = 16

        def gather_mul(i0, n, coef, add):
            def body(v, carry):
                w0 = v * (16 * UNROLL)
                for s in range(UNROLL):
                    o = w0 + s * 16
                    idvec = idb[pl.ds(o, 16)]
                    val = plsc.load_gather(rowb, [idvec]) * coef
                    if add:
                        val = val + acc[pl.ds(i0 + o, 16)]
                    acc[pl.ds(i0 + o, 16)] = val
                return carry
            lax.fori_loop(0, n // (16 * UNROLL), body, 0)

        def do_feature(c, ranges):
            u_c = u_v[pl.ds(c, 16)][0]
            lap_c = lap_v[pl.ds(c, 16)][0]
            pltpu.sync_copy(at_hbm.at[c], rowb)
            for i0, n in ranges:
                pltpu.sync_copy(idx_hbm.at[pl.ds(i0, n)], idb.at[pl.ds(0, n)])
                gather_mul(i0, n, u_c, False)
            pltpu.sync_copy(alt_hbm.at[c], rowb)
            for j, (i0, n) in enumerate(ranges[::-1]):
                if j > 0:
                    pltpu.sync_copy(idx_hbm.at[pl.ds(i0, n)],
                                    idb.at[pl.ds(0, n)])
                gather_mul(i0, n, lap_c, True)
            for i0, n in ranges:
                pltpu.sync_copy(acc.at[pl.ds(i0, n)],
                                out_hbm.at[c, pl.ds(i0, n)])

        for rnd in range(3):
            do_feature(wid + rnd * NW, [(0, HB), (HB, HB)])

        TAIL = D - 3 * NW
        QB = B // 4

        @pl.when(wid < 4 * TAIL)
        def _():
            c = 3 * NW + wid // 4
            q = wid % 4
            do_feature(c, [(q * QB, QB)])

    return sc_call


_SC_CALL = None


def kernel(x, data_id, frame, a_, b_, alpha_):
    global _SC_CALL
    if _SC_CALL is None:
        _SC_CALL = _build_sc_call()
    u = x[:, 0]
    lap = x[:, 2]
    idx = data_id.astype(jnp.int32)
    out_t = _SC_CALL(u, lap, idx, a_.T, alpha_.T)
    return out_t.T


# speedup vs baseline: 1.4800x; 1.0065x over previous
import functools

import jax
import jax.numpy as jnp
from jax import lax
from jax.experimental import pallas as pl
from jax.experimental.pallas import tpu as pltpu
from jax.experimental.pallas import tpu_sc as plsc

B = 16384
HB = B // 2
D = 99
N_DATASETS = 100000

NC, NS = 2, 16
NW = NC * NS
ROUNDS = (D + NW - 1) // NW


def _build_sc_call():
    mesh = plsc.VectorSubcoreMesh(
        core_axis_name="c", subcore_axis_name="s",
        num_cores=NC, num_subcores=NS)

    @functools.partial(
        pl.kernel,
        mesh=mesh,
        compiler_params=pltpu.CompilerParams(needs_layout_passes=False),
        out_type=jax.ShapeDtypeStruct((D, B), jnp.float32),
        scratch_types=[
            pltpu.VMEM((N_DATASETS,), jnp.float32),
            pltpu.VMEM((HB,), jnp.int32),
            pltpu.VMEM((B,), jnp.float32),
            pltpu.VMEM((D + 16,), jnp.float32),
            pltpu.VMEM((D + 16,), jnp.float32),
            pltpu.SemaphoreType.DMA,
        ],
    )
    def sc_call(u_hbm, lap_hbm, idx_hbm, at_hbm, alt_hbm, out_hbm,
                rowb, idb, acc, u_v, lap_v, osem):
        wid = lax.axis_index("s") * NC + lax.axis_index("c")
        pltpu.sync_copy(u_hbm, u_v.at[pl.ds(0, D)])
        pltpu.sync_copy(lap_hbm, lap_v.at[pl.ds(0, D)])

        UNROLL = 16

        def gather_mul(i0, n, coef, add):
            def body(v, carry):
                w0 = v * (16 * UNROLL)
                for s in range(UNROLL):
                    o = w0 + s * 16
                    idvec = idb[pl.ds(o, 16)]
                    val = plsc.load_gather(rowb, [idvec]) * coef
                    if add:
                        val = val + acc[pl.ds(i0 + o, 16)]
                    acc[pl.ds(i0 + o, 16)] = val
                return carry
            lax.fori_loop(0, n // (16 * UNROLL), body, 0)

        def do_feature(c, ranges, pending, sync_out):
            u_c = u_v[pl.ds(c, 16)][0]
            lap_c = lap_v[pl.ds(c, 16)][0]
            pltpu.sync_copy(at_hbm.at[c], rowb)
            for h in pending:
                h.wait()
            for i0, n in ranges:
                pltpu.sync_copy(idx_hbm.at[pl.ds(i0, n)], idb.at[pl.ds(0, n)])
                gather_mul(i0, n, u_c, False)
            pltpu.sync_copy(alt_hbm.at[c], rowb)
            for j, (i0, n) in enumerate(ranges[::-1]):
                if j > 0:
                    pltpu.sync_copy(idx_hbm.at[pl.ds(i0, n)],
                                    idb.at[pl.ds(0, n)])
                gather_mul(i0, n, lap_c, True)
            if sync_out:
                for i0, n in ranges:
                    pltpu.sync_copy(acc.at[pl.ds(i0, n)],
                                    out_hbm.at[c, pl.ds(i0, n)])
                return []
            return [pltpu.async_copy(acc.at[pl.ds(i0, n)],
                                     out_hbm.at[c, pl.ds(i0, n)], osem)
                    for i0, n in ranges]

        pending = []
        for rnd in range(3):
            pending = do_feature(wid + rnd * NW, [(0, HB), (HB, HB)],
                                 pending, False)
        for h in pending:
            h.wait()

        TAIL = D - 3 * NW
        QB = B // 4

        @pl.when(wid < 4 * TAIL)
        def _():
            c = 3 * NW + wid // 4
            q = wid % 4
            do_feature(c, [(q * QB, QB)], [], True)

    return sc_call


_SC_CALL = None


def kernel(x, data_id, frame, a_, b_, alpha_):
    global _SC_CALL
    if _SC_CALL is None:
        _SC_CALL = _build_sc_call()
    u = x[:, 0]
    lap = x[:, 2]
    idx = data_id.astype(jnp.int32)
    out_t = _SC_CALL(u, lap, idx, a_.T, alpha_.T)
    return out_t.T


# speedup vs baseline: 1.4818x; 1.0012x over previous
import functools

import jax
import jax.numpy as jnp
from jax import lax
from jax.experimental import pallas as pl
from jax.experimental.pallas import tpu as pltpu
from jax.experimental.pallas import tpu_sc as plsc

B = 16384
HB = B // 2
D = 99
N_DATASETS = 100000

NC, NS = 2, 16
NW = NC * NS


def _build_sc_call():
    mesh = plsc.VectorSubcoreMesh(
        core_axis_name="c", subcore_axis_name="s",
        num_cores=NC, num_subcores=NS)

    @functools.partial(
        pl.kernel,
        mesh=mesh,
        compiler_params=pltpu.CompilerParams(needs_layout_passes=False),
        out_type=jax.ShapeDtypeStruct((D, B), jnp.float32),
        scratch_types=[
            pltpu.VMEM((N_DATASETS,), jnp.float32),
            pltpu.VMEM((HB,), jnp.int32),
            pltpu.VMEM((B,), jnp.float32),
            pltpu.VMEM((D + 16,), jnp.float32),
            pltpu.VMEM((D + 16,), jnp.float32),
            pltpu.SemaphoreType.DMA,
        ],
    )
    def sc_call(u_hbm, lap_hbm, idx_hbm, at_hbm, alt_hbm, out_hbm,
                rowb, idb, acc, u_v, lap_v, osem):
        wid = lax.axis_index("s") * NC + lax.axis_index("c")
        pltpu.sync_copy(u_hbm, u_v.at[pl.ds(0, D)])
        pltpu.sync_copy(lap_hbm, lap_v.at[pl.ds(0, D)])

        UNROLL = 16

        def gather_mul(i0, n, coef, add):
            def body(v, carry):
                w0 = v * (16 * UNROLL)
                for s in range(UNROLL):
                    o = w0 + s * 16
                    idvec = idb[pl.ds(o, 16)]
                    val = plsc.load_gather(rowb, [idvec]) * coef
                    if add:
                        val = val + acc[pl.ds(i0 + o, 16)]
                    acc[pl.ds(i0 + o, 16)] = val
                return carry
            lax.fori_loop(0, n // (16 * UNROLL), body, 0)

        def do_feature(c, ranges, pending, sync_out):
            u_c = u_v[pl.ds(c, 16)][0]
            lap_c = lap_v[pl.ds(c, 16)][0]
            pltpu.sync_copy(at_hbm.at[c], rowb)
            for h in pending:
                h.wait()
            for i0, n in ranges:
                pltpu.sync_copy(idx_hbm.at[pl.ds(i0, n)], idb.at[pl.ds(0, n)])
                gather_mul(i0, n, u_c, False)
            pltpu.sync_copy(alt_hbm.at[c], rowb)
            for j, (i0, n) in enumerate(ranges[::-1]):
                if j > 0:
                    pltpu.sync_copy(idx_hbm.at[pl.ds(i0, n)],
                                    idb.at[pl.ds(0, n)])
                gather_mul(i0, n, lap_c, True)
            if sync_out:
                for i0, n in ranges:
                    pltpu.sync_copy(acc.at[pl.ds(i0, n)],
                                    out_hbm.at[c, pl.ds(i0, n)])
                return []
            return [pltpu.async_copy(acc.at[pl.ds(i0, n)],
                                     out_hbm.at[c, pl.ds(i0, n)], osem)
                    for i0, n in ranges]

        pending = []
        for rnd in range(3):
            pending = do_feature(wid + rnd * NW, [(0, HB), (HB, HB)],
                                 pending, False)
        for h in pending:
            h.wait()

        TAIL = D - 3 * NW
        QB = B // 4

        @pl.when(wid < 4 * TAIL)
        def _():
            c = 3 * NW + wid // 4
            q = wid % 4
            do_feature(c, [(q * QB, QB)], [], True)

    return sc_call


_SC_CALL = None


def kernel(x, data_id, frame, a_, b_, alpha_):
    global _SC_CALL
    if _SC_CALL is None:
        _SC_CALL = _build_sc_call()
    u = x[:, 0]
    lap = x[:, 2]
    idx = data_id.astype(jnp.int32)
    out_t = _SC_CALL(u, lap, idx, a_.T, alpha_.T)
    return out_t.T
